# baseline (speedup 1.0000x reference)
import functools

import jax
import jax.numpy as jnp
from jax import lax
from jax.experimental import pallas as pl
from jax.experimental.pallas import tpu as pltpu
from jax.experimental.pallas import tpu_sc as plsc

_THRESH = 0.01

_NC = 2
_NS = 16
_NW = _NC * _NS
_LANES = 16

_N = 2 * 4096 * 4096
_E = _N // _NW
_CHUNK = 16384
_ITERS = _E // _CHUNK


def _sc_partials(o_flat, t_flat, a_flat):
    mesh = plsc.VectorSubcoreMesh(
        core_axis_name="c", subcore_axis_name="s",
        num_cores=_NC, num_subcores=_NS)

    @functools.partial(
        pl.kernel,
        out_type=(
            jax.ShapeDtypeStruct((_NW, _LANES), jnp.float32),
            jax.ShapeDtypeStruct((_NW, _LANES), jnp.float32),
        ),
        mesh=mesh,
        scratch_types=[
            pltpu.VMEM((_CHUNK,), jnp.float32),
            pltpu.VMEM((_CHUNK,), jnp.float32),
            pltpu.VMEM((_CHUNK,), jnp.float32),
            pltpu.VMEM((_LANES,), jnp.float32),
            pltpu.VMEM((_LANES,), jnp.float32),
        ],
    )
    def k(o_hbm, t_hbm, a_hbm, num_hbm, den_hbm,
          o_buf, t_buf, a_buf, num_stage, den_stage):
        wid = lax.axis_index("s") * _NC + lax.axis_index("c")
        base = wid * _E

        def chunk_body(it, carry):
            num, den = carry
            start = base + it * _CHUNK
            pltpu.sync_copy(o_hbm.at[pl.ds(start, _CHUNK)], o_buf)
            pltpu.sync_copy(t_hbm.at[pl.ds(start, _CHUNK)], t_buf)
            pltpu.sync_copy(a_hbm.at[pl.ds(start, _CHUNK)], a_buf)

            def vec_body(i, c):
                n, d = c
                o = o_buf[pl.ds(i * _LANES, _LANES)]
                t = t_buf[pl.ds(i * _LANES, _LANES)]
                a = a_buf[pl.ds(i * _LANES, _LANES)]
                m = jnp.abs(a) > _THRESH
                diff = o - t
                n = n + jnp.where(m, diff * diff, 0.0)
                d = d + jnp.where(m, 1.0, 0.0)
                return (n, d)

            return lax.fori_loop(0, _CHUNK // _LANES, vec_body, (num, den))

        zero = jnp.zeros((_LANES,), jnp.float32)
        num, den = lax.fori_loop(0, _ITERS, chunk_body, (zero, zero))
        num_stage[...] = num
        den_stage[...] = den
        pltpu.sync_copy(num_stage, num_hbm.at[wid])
        pltpu.sync_copy(den_stage, den_hbm.at[wid])

    return k(o_flat, t_flat, a_flat)


def _tc_combine(num_parts, den_parts):
    def body(num_ref, den_ref, out_ref):
        num = jnp.sum(num_ref[...])
        den = jnp.sum(den_ref[...])
        out_ref[0, 0] = num / den

    out = pl.pallas_call(
        body,
        out_shape=jax.ShapeDtypeStruct((1, 1), jnp.float32),
        out_specs=pl.BlockSpec(memory_space=pltpu.SMEM),
    )(num_parts, den_parts)
    return out[0, 0]


def kernel(output, target, analytic):
    o_flat = output.reshape(_N)
    t_flat = target.reshape(_N)
    a_flat = analytic.reshape(_N)
    num_parts, den_parts = _sc_partials(o_flat, t_flat, a_flat)
    return _tc_combine(num_parts, den_parts)


# speedup vs baseline: 1.4079x; 1.4079x over previous
import functools

import jax
import jax.numpy as jnp
from jax import lax
from jax.experimental import pallas as pl
from jax.experimental.pallas import tpu as pltpu
from jax.experimental.pallas import tpu_sc as plsc

_THRESH = 0.01

_NC = 2
_NS = 16
_NW = _NC * _NS
_LANES = 16

_N = 2 * 4096 * 4096
_E = _N // _NW
_CHUNK = 16384
_ITERS = _E // _CHUNK


_UNROLL = 8


def _sc_partials(o_flat, t_flat, a_flat):
    mesh = plsc.VectorSubcoreMesh(
        core_axis_name="c", subcore_axis_name="s",
        num_cores=_NC, num_subcores=_NS)

    @functools.partial(
        pl.kernel,
        out_type=(
            jax.ShapeDtypeStruct((_NW, _LANES), jnp.float32),
            jax.ShapeDtypeStruct((_NW, _LANES), jnp.float32),
        ),
        mesh=mesh,
        scratch_types=[
            pltpu.VMEM((2, _CHUNK), jnp.float32),
            pltpu.VMEM((2, _CHUNK), jnp.float32),
            pltpu.VMEM((2, _CHUNK), jnp.float32),
            pltpu.VMEM((_LANES,), jnp.float32),
            pltpu.VMEM((_LANES,), jnp.float32),
            pltpu.SemaphoreType.DMA,
            pltpu.SemaphoreType.DMA,
        ],
    )
    def k(o_hbm, t_hbm, a_hbm, num_hbm, den_hbm,
          o_buf, t_buf, a_buf, num_stage, den_stage, sem0, sem1):
        wid = lax.axis_index("s") * _NC + lax.axis_index("c")
        base = wid * _E
        last_start = base + (_ITERS - 1) * _CHUNK

        def fire(chunk_idx, slot, sem):
            start = jnp.minimum(base + chunk_idx * _CHUNK, last_start)
            pltpu.async_copy(o_hbm.at[pl.ds(start, _CHUNK)], o_buf.at[slot], sem)
            pltpu.async_copy(t_hbm.at[pl.ds(start, _CHUNK)], t_buf.at[slot], sem)
            pltpu.async_copy(a_hbm.at[pl.ds(start, _CHUNK)], a_buf.at[slot], sem)

        def drain(slot, sem):
            pltpu.make_async_copy(o_hbm.at[pl.ds(base, _CHUNK)], o_buf.at[slot], sem).wait()
            pltpu.make_async_copy(t_hbm.at[pl.ds(base, _CHUNK)], t_buf.at[slot], sem).wait()
            pltpu.make_async_copy(a_hbm.at[pl.ds(base, _CHUNK)], a_buf.at[slot], sem).wait()

        def compute(slot, accs):
            o_s, t_s, a_s = o_buf.at[slot], t_buf.at[slot], a_buf.at[slot]

            def vec_block(i, accs):
                accs = list(accs)
                off = i * (_UNROLL * _LANES)
                for u in range(_UNROLL):
                    sl = pl.ds(off + u * _LANES, _LANES)
                    o = o_s[sl]
                    t = t_s[sl]
                    a = a_s[sl]
                    m = jnp.abs(a) > _THRESH
                    diff = o - t
                    n, d = accs[u]
                    n = n + jnp.where(m, diff * diff, 0.0)
                    d = d + jnp.where(m, 1.0, 0.0)
                    accs[u] = (n, d)
                return tuple(accs)

            return lax.fori_loop(
                0, _CHUNK // (_UNROLL * _LANES), vec_block, accs)

        zero = jnp.zeros((_LANES,), jnp.float32)
        accs = tuple((zero, zero) for _ in range(_UNROLL))
        fire(0, 0, sem0)

        def outer(j, accs):
            b = 2 * j
            fire(b + 1, 1, sem1)
            drain(0, sem0)
            accs = compute(0, accs)
            fire(b + 2, 0, sem0)
            drain(1, sem1)
            return compute(1, accs)

        accs = lax.fori_loop(0, _ITERS // 2, outer, accs)
        drain(0, sem0)

        num = functools.reduce(lambda x, y: x + y, [a[0] for a in accs])
        den = functools.reduce(lambda x, y: x + y, [a[1] for a in accs])
        num_stage[...] = num
        den_stage[...] = den
        pltpu.sync_copy(num_stage, num_hbm.at[wid])
        pltpu.sync_copy(den_stage, den_hbm.at[wid])

    return k(o_flat, t_flat, a_flat)


def _tc_combine(num_parts, den_parts):
    def body(num_ref, den_ref, out_ref):
        num = jnp.sum(num_ref[...])
        den = jnp.sum(den_ref[...])
        out_ref[0, 0] = num / den

    out = pl.pallas_call(
        body,
        out_shape=jax.ShapeDtypeStruct((1, 1), jnp.float32),
        out_specs=pl.BlockSpec(memory_space=pltpu.SMEM),
    )(num_parts, den_parts)
    return out[0, 0]


def kernel(output, target, analytic):
    o_flat = output.reshape(_N)
    t_flat = target.reshape(_N)
    a_flat = analytic.reshape(_N)
    num_parts, den_parts = _sc_partials(o_flat, t_flat, a_flat)
    return _tc_combine(num_parts, den_parts)
